# Initial kernel scaffold; baseline (speedup 1.0000x reference)
import jax
import jax.numpy as jnp
from jax.experimental import pallas as pl


def kernel(input_seq, edge_index, features, emb_table, W_conv3, b_conv3, W_conv4, b_conv4, W_conv5, b_conv5, W_fc, b_fc, W_g1, b_g1, W_g2, b_g2):
    raise NotImplementedError("write your pallas kernel here")


# speedup vs baseline: 15.6330x; 15.6330x over previous
import functools

import jax
import jax.numpy as jnp
from jax import lax
from jax.experimental import pallas as pl
from jax.experimental.pallas import tpu as pltpu
from jax.experimental.pallas import tpu_sc as plsc

N_NODES = 10000
NC, NS = 2, 16
NW = NC * NS
ACC_ROWS = 10240
E_EDGES = 320000
CHUNK = 128
EDGE_CHUNKS_PER_TILE = 80
PAD_EDGES = NW * EDGE_CHUNKS_PER_TILE * CHUNK
EMB_D = 256
B, L = 256, 512
TOK_CHUNKS_PER_TILE = (B * L) // (NW * CHUNK)


@functools.cache
def _get_mesh():
    return plsc.VectorSubcoreMesh(
        core_axis_name="c", subcore_axis_name="s",
        num_cores=NC, num_subcores=NS)


@functools.cache
def _make_emb_gather():
    @functools.partial(
        pl.kernel,
        out_type=jax.ShapeDtypeStruct((B * L, EMB_D), jnp.float32),
        mesh=_get_mesh(),
        scratch_types=[
            pltpu.VMEM((TOK_CHUNKS_PER_TILE, CHUNK), jnp.int32),
            pltpu.VMEM((CHUNK, EMB_D), jnp.float32),
            pltpu.SemaphoreType.DMA,
        ],
    )
    def _emb_gather(tok_hbm, table_hbm, out_hbm, idx_v, rows_v, sem):
        c = lax.axis_index("c")
        s = lax.axis_index("s")
        wid = s * NC + c
        pltpu.sync_copy(tok_hbm.at[pl.ds(wid * TOK_CHUNKS_PER_TILE,
                                         TOK_CHUNKS_PER_TILE)], idx_v)

        def chunk(j, carry):
            pltpu.async_copy(table_hbm.at[idx_v.at[j]], rows_v, sem).wait()
            pltpu.sync_copy(
                rows_v,
                out_hbm.at[pl.ds((wid * TOK_CHUNKS_PER_TILE + j) * CHUNK,
                                 CHUNK)])
            return carry

        lax.fori_loop(0, TOK_CHUNKS_PER_TILE, chunk, 0)

    return _emb_gather


@functools.cache
def _make_segsum(n_tables):
    out_type = tuple(
        jax.ShapeDtypeStruct((NC, ACC_ROWS, 128), jnp.float32)
        for _ in range(n_tables))
    if n_tables == 1:
        out_type = out_type[0]
    scratch = [
        pltpu.VMEM((EDGE_CHUNKS_PER_TILE, CHUNK), jnp.int32),
        pltpu.VMEM((EDGE_CHUNKS_PER_TILE, CHUNK), jnp.int32),
        pltpu.VMEM((CHUNK, 128), jnp.float32),
        pltpu.VMEM_SHARED((ACC_ROWS, 128), jnp.float32),
        pltpu.SemaphoreType.DMA,
    ]

    @functools.partial(pl.kernel, out_type=out_type, mesh=_get_mesh(),
                       scratch_types=scratch)
    def seg(src_hbm, dst_hbm, zero_hbm, *rest):
        tables = rest[:n_tables]
        outs = rest[n_tables:2 * n_tables]
        sidx, didx, rows, acc, sem = rest[2 * n_tables:]
        c = lax.axis_index("c")
        s = lax.axis_index("s")
        wid = s * NC + c
        pltpu.sync_copy(
            src_hbm.at[pl.ds(wid * EDGE_CHUNKS_PER_TILE,
                             EDGE_CHUNKS_PER_TILE)], sidx)
        pltpu.sync_copy(
            dst_hbm.at[pl.ds(wid * EDGE_CHUNKS_PER_TILE,
                             EDGE_CHUNKS_PER_TILE)], didx)
        for t in range(n_tables):
            for blk in range(5):
                pltpu.sync_copy(zero_hbm,
                                acc.at[pl.ds((s * 5 + blk) * 128, 128)])
            plsc.subcore_barrier()

            def chunk(j, carry):
                pltpu.async_copy(tables[t].at[sidx.at[j]], rows, sem).wait()
                pltpu.sync_copy(rows, acc.at[didx.at[j]], add=True)
                return carry

            lax.fori_loop(0, EDGE_CHUNKS_PER_TILE, chunk, 0)
            plsc.subcore_barrier()
            pltpu.sync_copy(acc.at[pl.ds(s * 640, 640)],
                            outs[t].at[c, pl.ds(s * 640, 640)])
            if t + 1 < n_tables:
                plsc.subcore_barrier()

    return seg


def _lmax_body(seq_ref, o_ref):
    cnt = jnp.sum((seq_ref[...] != 0).astype(jnp.int32), axis=1)
    o_ref[0, 0] = jnp.max(cnt)


def _conv_body(e_ref, w_ref, b_ref, lm_ref, o_ref):
    p = jnp.dot(e_ref[...], w_ref[...], preferred_element_type=jnp.float32)
    lmax = lm_ref[0, 0]
    pos = lax.broadcasted_iota(jnp.int32, (L, 128), 0)
    off = 0
    for ki, k in enumerate((3, 4, 5)):
        acc = p[:, off * 128:(off + 1) * 128]
        for j in range(1, k):
            col = (off + j) * 128
            blkj = p[:, col:col + 128]
            acc = acc + jnp.concatenate(
                [blkj[j:, :], jnp.zeros((j, 128), jnp.float32)], axis=0)
        off += k
        cval = jnp.maximum(acc + b_ref[0, ki * 128:(ki + 1) * 128][None, :],
                           0.0)
        cval = jnp.where(pos <= lmax - k, cval, -jnp.inf)
        o_ref[0, 0, ki * 128:(ki + 1) * 128] = jnp.max(cval, axis=0)


def _fc_body(x_ref, w_ref, b_ref, o_ref):
    o_ref[...] = jax.nn.sigmoid(
        jnp.dot(x_ref[...], w_ref[...], preferred_element_type=jnp.float32)
        + b_ref[...])


def _gcn1_body(p0_ref, p1_ref, w_ref, b_ref, oa_ref, ob_ref):
    h = p0_ref[...] + p1_ref[...]
    h1 = jnp.maximum(
        jnp.dot(h, w_ref[...], preferred_element_type=jnp.float32)
        + b_ref[...], 0.0)
    oa_ref[...] = h1[:, :128]
    ob_ref[...] = h1[:, 128:]


def _final_body(qa0_ref, qa1_ref, qb0_ref, qb1_ref, wa_ref, wb_ref, b_ref,
                xc_ref, o_ref):
    ha = qa0_ref[...] + qa1_ref[...]
    hb = qb0_ref[...] + qb1_ref[...]
    label = (jnp.dot(ha, wa_ref[...], preferred_element_type=jnp.float32)
             + jnp.dot(hb, wb_ref[...], preferred_element_type=jnp.float32)
             + b_ref[...])
    t = lax.dot_general(xc_ref[...], label, (((1,), (1,)), ((), ())),
                        preferred_element_type=jnp.float32)
    o_ref[0] = jax.nn.sigmoid(t)


def kernel(input_seq, edge_index, features, emb_table, W_conv3, b_conv3,
           W_conv4, b_conv4, W_conv5, b_conv5, W_fc, b_fc, W_g1, b_g1,
           W_g2, b_g2):
    f32 = jnp.float32
    i32 = jnp.int32

    tok = input_seq.astype(i32).reshape(-1, CHUNK)
    table = jnp.pad(emb_table.astype(f32), ((0, 0), (0, EMB_D - 200)))
    src = edge_index[0].astype(i32)
    dst = edge_index[1].astype(i32)
    npad = PAD_EDGES - E_EDGES
    srcp = jnp.concatenate([src, jnp.zeros((npad,), i32)]).reshape(-1, CHUNK)
    dstp = jnp.concatenate([dst, jnp.full((npad,), N_NODES, i32)]
                           ).reshape(-1, CHUNK)
    zeros128 = jnp.zeros((CHUNK, 128), f32)
    ws = []
    for W in (W_conv3, W_conv4, W_conv5):
        wt = jnp.transpose(W[:, 0], (1, 2, 0))
        ws.extend([wt[j] for j in range(wt.shape[0])])
    w_all = jnp.pad(jnp.concatenate(ws, axis=1),
                    ((0, EMB_D - 200), (0, 0)))
    bcat = jnp.concatenate([b_conv3, b_conv4, b_conv5]).reshape(1, 384)

    lmax = pl.pallas_call(
        _lmax_body,
        out_shape=jax.ShapeDtypeStruct((1, 1), i32),
        in_specs=[pl.BlockSpec((B, L), lambda: (0, 0))],
        out_specs=pl.BlockSpec(memory_space=pltpu.SMEM),
    )(input_seq.astype(i32))

    emb = _make_emb_gather()(tok, table)

    xcat3 = pl.pallas_call(
        _conv_body,
        grid=(B,),
        out_shape=jax.ShapeDtypeStruct((B, 1, 384), f32),
        in_specs=[
            pl.BlockSpec((L, EMB_D), lambda i: (i, 0)),
            pl.BlockSpec((EMB_D, 1536), lambda i: (0, 0)),
            pl.BlockSpec((1, 384), lambda i: (0, 0)),
            pl.BlockSpec((1, 1), lambda i: (0, 0)),
        ],
        out_specs=pl.BlockSpec((1, 1, 384), lambda i: (i, 0, 0)),
    )(emb, w_all, bcat, lmax)

    x_cnn = pl.pallas_call(
        _fc_body,
        out_shape=jax.ShapeDtypeStruct((B, 256), f32),
        in_specs=[
            pl.BlockSpec((B, 384), lambda: (0, 0)),
            pl.BlockSpec((384, 256), lambda: (0, 0)),
            pl.BlockSpec((1, 256), lambda: (0, 0)),
        ],
        out_specs=pl.BlockSpec((B, 256), lambda: (0, 0)),
    )(xcat3.reshape(B, 384), W_fc, b_fc.reshape(1, 256))

    p = _make_segsum(1)(srcp, dstp, zeros128, features)

    h1a, h1b = pl.pallas_call(
        _gcn1_body,
        grid=(10,),
        out_shape=(jax.ShapeDtypeStruct((N_NODES, 128), f32),
                   jax.ShapeDtypeStruct((N_NODES, 128), f32)),
        in_specs=[
            pl.BlockSpec((1000, 128), lambda i: (i, 0)),
            pl.BlockSpec((1000, 128), lambda i: (i, 0)),
            pl.BlockSpec((128, 256), lambda i: (0, 0)),
            pl.BlockSpec((1, 256), lambda i: (0, 0)),
        ],
        out_specs=(pl.BlockSpec((1000, 128), lambda i: (i, 0)),
                   pl.BlockSpec((1000, 128), lambda i: (i, 0))),
    )(p[0], p[1], W_g1, b_g1.reshape(1, 256))

    qa, qb = _make_segsum(2)(srcp, dstp, zeros128, h1a, h1b)

    out = pl.pallas_call(
        _final_body,
        grid=(10,),
        out_shape=jax.ShapeDtypeStruct((10, B, 1000), f32),
        in_specs=[
            pl.BlockSpec((1000, 128), lambda i: (i, 0)),
            pl.BlockSpec((1000, 128), lambda i: (i, 0)),
            pl.BlockSpec((1000, 128), lambda i: (i, 0)),
            pl.BlockSpec((1000, 128), lambda i: (i, 0)),
            pl.BlockSpec((128, 256), lambda i: (0, 0)),
            pl.BlockSpec((128, 256), lambda i: (0, 0)),
            pl.BlockSpec((1, 256), lambda i: (0, 0)),
            pl.BlockSpec((B, 256), lambda i: (0, 0)),
        ],
        out_specs=pl.BlockSpec((1, B, 1000), lambda i: (i, 0, 0)),
    )(qa[0], qa[1], qb[0], qb[1], W_g2[:128], W_g2[128:],
      b_g2.reshape(1, 256), x_cnn)

    return out.transpose(1, 0, 2).reshape(B, N_NODES)
